# Initial kernel scaffold; baseline (speedup 1.0000x reference)
import jax
import jax.numpy as jnp
from jax.experimental import pallas as pl


def kernel(x, edge_index, W1, b1, W2, b2, W3, b3, Wc, bc):
    raise NotImplementedError("write your pallas kernel here")


# speedup vs baseline: 82.3604x; 82.3604x over previous
import functools

import jax
import jax.numpy as jnp
from jax import lax
from jax.experimental import pallas as pl
from jax.experimental.pallas import tpu as pltpu
from jax.experimental.pallas import tpu_sc as plsc

N = 10000
NPAD = 10240
E = 320000
NCHUNK = 32
EPC = E // NCHUNK
LANES = 16

_MESH = plsc.VectorSubcoreMesh(core_axis_name="c", subcore_axis_name="s")
_SC_PARAMS = pltpu.CompilerParams(needs_layout_passes=False)


@functools.partial(
    pl.kernel,
    out_type=jax.ShapeDtypeStruct((NCHUNK, NPAD), jnp.float32),
    mesh=_MESH,
    compiler_params=_SC_PARAMS,
    scratch_types=[
        pltpu.VMEM((NPAD,), jnp.float32),
        pltpu.VMEM((EPC,), jnp.int32),
    ],
)
def _deg_sc(col_hbm, out_hbm, acc_v, col_v):
    cid = lax.axis_index("c")
    sid = lax.axis_index("s")
    wid = cid * 16 + sid
    base = pl.multiple_of(wid * EPC, 8)
    pltpu.sync_copy(col_hbm.at[pl.ds(base, EPC)], col_v)

    zero = jnp.zeros((LANES,), jnp.float32)

    @pl.loop(0, NPAD // LANES)
    def _(i):
        acc_v[pl.ds(i * LANES, LANES)] = zero

    ones = jnp.ones((LANES,), jnp.float32)

    @pl.loop(0, EPC // LANES)
    def _(i):
        c = col_v[pl.ds(i * LANES, LANES)]
        plsc.addupdate_scatter(acc_v, [c], ones)

    pltpu.sync_copy(acc_v, out_hbm.at[wid])


def _make_agg(F):
    @functools.partial(
        pl.kernel,
        out_type=jax.ShapeDtypeStruct((NCHUNK, F, NPAD), jnp.float32),
        mesh=_MESH,
        compiler_params=_SC_PARAMS,
        scratch_types=[
            pltpu.VMEM((F, NPAD), jnp.float32),
            pltpu.VMEM((F, NPAD), jnp.float32),
            pltpu.VMEM((EPC,), jnp.int32),
            pltpu.VMEM((EPC,), jnp.int32),
        ],
    )
    def _agg_sc(hs_hbm, row_hbm, col_hbm, out_hbm, hs_v, acc_v, row_v, col_v):
        cid = lax.axis_index("c")
        sid = lax.axis_index("s")
        wid = cid * 16 + sid
        base = pl.multiple_of(wid * EPC, 8)
        pltpu.sync_copy(hs_hbm, hs_v)
        pltpu.sync_copy(row_hbm.at[pl.ds(base, EPC)], row_v)
        pltpu.sync_copy(col_hbm.at[pl.ds(base, EPC)], col_v)

        zero = jnp.zeros((LANES,), jnp.float32)

        @pl.loop(0, NPAD // LANES)
        def _(i):
            for f in range(F):
                acc_v[f, pl.ds(i * LANES, LANES)] = zero

        fidx = [jnp.full((LANES,), f, jnp.int32) for f in range(F)]

        @pl.loop(0, EPC // LANES)
        def _(i):
            r = row_v[pl.ds(i * LANES, LANES)]
            c = col_v[pl.ds(i * LANES, LANES)]
            for f in range(F):
                v = plsc.load_gather(hs_v, [fidx[f], r])
                plsc.addupdate_scatter(acc_v, [fidx[f], c], v)

        pltpu.sync_copy(acc_v, out_hbm.at[wid])

    return _agg_sc


_agg4_sc = _make_agg(4)
_agg2_sc = _make_agg(2)


def _tc_a(x, W1, degp):
    def body(x_ref, w_ref, degp_ref, dis_ref, hs_ref):
        deg = jnp.sum(degp_ref[...], axis=0) + 1.0
        dis = lax.rsqrt(deg)
        dis_ref[...] = dis[None, :]
        h0t = lax.dot_general(
            w_ref[...], x_ref[...], (((0,), (1,)), ((), ())),
            preferred_element_type=jnp.float32)
        hs_ref[...] = jnp.zeros_like(hs_ref)
        hs_ref[:, :N] = h0t * dis[None, :N]

    return pl.pallas_call(
        body,
        out_shape=(
            jax.ShapeDtypeStruct((1, NPAD), jnp.float32),
            jax.ShapeDtypeStruct((4, NPAD), jnp.float32),
        ),
    )(x, W1, degp)


def _tc_mid(p, hs, dis, W, b):
    F_out = W.shape[1]

    def body(p_ref, hs_ref, dis_ref, w_ref, b_ref, o_ref):
        dis2 = dis_ref[...]
        agg = jnp.sum(p_ref[...], axis=0) + hs_ref[...]
        t = jnp.tanh(dis2 * agg + b_ref[...])
        o_ref[...] = lax.dot_general(
            w_ref[...], t, (((0,), (0,)), ((), ())),
            preferred_element_type=jnp.float32) * dis2

    return pl.pallas_call(
        body,
        out_shape=jax.ShapeDtypeStruct((F_out, NPAD), jnp.float32),
    )(p, hs, dis, W, b)


def _tc_final(p, hs, dis, b3, Wc, bc):
    def body(p_ref, hs_ref, dis_ref, b3_ref, wc_ref, bc_ref, h3_ref, out_ref):
        dis2 = dis_ref[...]
        agg = jnp.sum(p_ref[...], axis=0) + hs_ref[...]
        h3 = jnp.tanh(dis2 * agg + b3_ref[...])
        h3_ref[...] = h3
        out_ref[...] = lax.dot_general(
            wc_ref[...], h3, (((0,), (0,)), ((), ())),
            preferred_element_type=jnp.float32) + bc_ref[...]

    return pl.pallas_call(
        body,
        out_shape=(
            jax.ShapeDtypeStruct((2, NPAD), jnp.float32),
            jax.ShapeDtypeStruct((16, NPAD), jnp.float32),
        ),
    )(p, hs, dis, b3, Wc, bc)


def kernel(x, edge_index, W1, b1, W2, b2, W3, b3, Wc, bc):
    row = edge_index[0]
    col = edge_index[1]

    degp = _deg_sc(col)
    dis, hs0t = _tc_a(x, W1, degp)

    p1 = _agg4_sc(hs0t, row, col)
    hs1t = _tc_mid(p1, hs0t, dis, W2, b1.reshape(4, 1))

    p2 = _agg4_sc(hs1t, row, col)
    hs2t = _tc_mid(p2, hs1t, dis, W3, b2.reshape(4, 1))

    p3 = _agg2_sc(hs2t, row, col)
    h3t, outt = _tc_final(p3, hs2t, dis, b3.reshape(2, 1), Wc,
                          bc.reshape(16, 1))

    return outt[:, :N].T, h3t[:, :N].T


# speedup vs baseline: 141.1880x; 1.7143x over previous
import functools

import jax
import jax.numpy as jnp
from jax import lax
from jax.experimental import pallas as pl
from jax.experimental.pallas import tpu as pltpu
from jax.experimental.pallas import tpu_sc as plsc

N = 10000
NPAD = 10240
E = 320000
NCHUNK = 32
EPC = E // NCHUNK
LANES = 16

_MESH = plsc.VectorSubcoreMesh(core_axis_name="c", subcore_axis_name="s")
_SC_PARAMS = pltpu.CompilerParams(needs_layout_passes=False)


@functools.partial(
    pl.kernel,
    out_type=jax.ShapeDtypeStruct((NCHUNK, NPAD), jnp.float32),
    mesh=_MESH,
    compiler_params=_SC_PARAMS,
    scratch_types=[
        pltpu.VMEM((NPAD,), jnp.float32),
        pltpu.VMEM((EPC,), jnp.int32),
        pltpu.SemaphoreType.DMA,
    ],
)
def _deg_sc(col_hbm, out_hbm, acc_v, col_v, sem):
    cid = lax.axis_index("c")
    sid = lax.axis_index("s")
    wid = cid * 16 + sid
    base = pl.multiple_of(wid * EPC, 8)
    cp = pltpu.async_copy(col_hbm.at[pl.ds(base, EPC)], col_v, sem)

    zero = jnp.zeros((LANES,), jnp.float32)

    @functools.partial(plsc.parallel_loop, 0, NPAD // LANES, unroll=8)
    def _(i):
        acc_v[pl.ds(i * LANES, LANES)] = zero

    cp.wait()
    ones = jnp.ones((LANES,), jnp.float32)

    @functools.partial(plsc.parallel_loop, 0, EPC // LANES, unroll=8)
    def _(i):
        c = col_v[pl.ds(i * LANES, LANES)]
        plsc.addupdate_scatter(acc_v, [c], ones)

    pltpu.sync_copy(acc_v, out_hbm.at[wid])


def _make_agg(F):
    @functools.partial(
        pl.kernel,
        out_type=jax.ShapeDtypeStruct((NCHUNK, F, NPAD), jnp.float32),
        mesh=_MESH,
        compiler_params=_SC_PARAMS,
        scratch_types=[
            pltpu.VMEM((F, NPAD), jnp.float32),
            pltpu.VMEM((F, NPAD), jnp.float32),
            pltpu.VMEM((EPC,), jnp.int32),
            pltpu.VMEM((EPC,), jnp.int32),
            pltpu.SemaphoreType.DMA,
        ],
    )
    def _agg_sc(hs_hbm, row_hbm, col_hbm, out_hbm, hs_v, acc_v, row_v, col_v,
                sem):
        cid = lax.axis_index("c")
        sid = lax.axis_index("s")
        wid = cid * 16 + sid
        base = pl.multiple_of(wid * EPC, 8)
        cp0 = pltpu.async_copy(hs_hbm, hs_v, sem)
        cp1 = pltpu.async_copy(row_hbm.at[pl.ds(base, EPC)], row_v, sem)
        cp2 = pltpu.async_copy(col_hbm.at[pl.ds(base, EPC)], col_v, sem)

        zero = jnp.zeros((LANES,), jnp.float32)

        @functools.partial(plsc.parallel_loop, 0, NPAD // LANES, unroll=8)
        def _(i):
            for f in range(F):
                acc_v[f, pl.ds(i * LANES, LANES)] = zero

        cp0.wait()
        cp1.wait()
        cp2.wait()

        fidx = [jnp.full((LANES,), f, jnp.int32) for f in range(F)]

        @functools.partial(plsc.parallel_loop, 0, EPC // LANES, unroll=4)
        def _(i):
            r = row_v[pl.ds(i * LANES, LANES)]
            c = col_v[pl.ds(i * LANES, LANES)]
            for f in range(F):
                v = plsc.load_gather(hs_v, [fidx[f], r])
                plsc.addupdate_scatter(acc_v, [fidx[f], c], v)

        pltpu.sync_copy(acc_v, out_hbm.at[wid])

    return _agg_sc


_agg4_sc = _make_agg(4)
_agg2_sc = _make_agg(2)


def _tc_a(x, W1, degp):
    def body(x_ref, w_ref, degp_ref, dis_ref, hs_ref):
        deg = jnp.sum(degp_ref[...], axis=0) + 1.0
        dis = lax.rsqrt(deg)
        dis_ref[...] = dis[None, :]
        h0t = lax.dot_general(
            w_ref[...], x_ref[...], (((0,), (1,)), ((), ())),
            preferred_element_type=jnp.float32)
        hs_ref[...] = jnp.zeros_like(hs_ref)
        hs_ref[:, :N] = h0t * dis[None, :N]

    return pl.pallas_call(
        body,
        out_shape=(
            jax.ShapeDtypeStruct((1, NPAD), jnp.float32),
            jax.ShapeDtypeStruct((4, NPAD), jnp.float32),
        ),
    )(x, W1, degp)


def _tc_mid(p, hs, dis, W, b):
    F_out = W.shape[1]

    def body(p_ref, hs_ref, dis_ref, w_ref, b_ref, o_ref):
        dis2 = dis_ref[...]
        agg = jnp.sum(p_ref[...], axis=0) + hs_ref[...]
        t = jnp.tanh(dis2 * agg + b_ref[...])
        o_ref[...] = lax.dot_general(
            w_ref[...], t, (((0,), (0,)), ((), ())),
            preferred_element_type=jnp.float32) * dis2

    return pl.pallas_call(
        body,
        out_shape=jax.ShapeDtypeStruct((F_out, NPAD), jnp.float32),
    )(p, hs, dis, W, b)


def _tc_final(p, hs, dis, b3, Wc, bc):
    def body(p_ref, hs_ref, dis_ref, b3_ref, wc_ref, bc_ref, h3_ref, out_ref):
        dis2 = dis_ref[...]
        agg = jnp.sum(p_ref[...], axis=0) + hs_ref[...]
        h3 = jnp.tanh(dis2 * agg + b3_ref[...])
        h3_ref[...] = h3
        out_ref[...] = lax.dot_general(
            wc_ref[...], h3, (((0,), (0,)), ((), ())),
            preferred_element_type=jnp.float32) + bc_ref[...]

    return pl.pallas_call(
        body,
        out_shape=(
            jax.ShapeDtypeStruct((2, NPAD), jnp.float32),
            jax.ShapeDtypeStruct((16, NPAD), jnp.float32),
        ),
    )(p, hs, dis, b3, Wc, bc)


def kernel(x, edge_index, W1, b1, W2, b2, W3, b3, Wc, bc):
    row = edge_index[0]
    col = edge_index[1]

    degp = _deg_sc(col)
    dis, hs0t = _tc_a(x, W1, degp)

    p1 = _agg4_sc(hs0t, row, col)
    hs1t = _tc_mid(p1, hs0t, dis, W2, b1.reshape(4, 1))

    p2 = _agg4_sc(hs1t, row, col)
    hs2t = _tc_mid(p2, hs1t, dis, W3, b2.reshape(4, 1))

    p3 = _agg2_sc(hs2t, row, col)
    h3t, outt = _tc_final(p3, hs2t, dis, b3.reshape(2, 1), Wc,
                          bc.reshape(16, 1))

    return outt[:, :N].T, h3t[:, :N].T
